# Initial kernel scaffold; baseline (speedup 1.0000x reference)
import jax
import jax.numpy as jnp
from jax.experimental import pallas as pl


def kernel(ent_feature_embed, ent_adj_ganea, ent_adj_w2v, ent_adj_transE, cand_mask_pad, mask, sim_M, diag_val_weight, W_gcn, b_gcn, W_out, b_out):
    raise NotImplementedError("write your pallas kernel here")


# speedup vs baseline: 5.6452x; 5.6452x over previous
import functools

import jax
import jax.numpy as jnp
from jax.experimental import pallas as pl
from jax.experimental.pallas import tpu as pltpu


def _gcn_body(emb_ref, g_ref, w_ref, t_ref, rmask_ref, cmask_t_ref, sim_ref,
              dvw_ref, wg_ref, bg_ref, wo_ref, bo_ref, omask_ref,
              out_ref, s_scr, h_scr, *, B, N):
    i = pl.program_id(0)
    b = pl.program_id(1)
    row0 = b * B

    h_full = jnp.where(i == 0, emb_ref[...], h_scr[...])
    hb = jnp.where(i == 0, emb_ref[pl.ds(row0, B), :],
                   h_scr[pl.ds(row0, B), :])

    l1 = jnp.dot(hb, sim_ref[...])
    l2 = jax.lax.dot_general(l1, h_full,
                             dimension_numbers=(((1,), (1,)), ((), ())))

    @pl.when(i == 0)
    def _():
        s_scr[pl.ds(row0, B), :] = g_ref[...] + w_ref[...] + t_ref[...]

    s_blk = s_scr[pl.ds(row0, B), :]

    a = jax.nn.sigmoid(0.25 * (l2 + s_blk))
    row_ids = row0 + jax.lax.broadcasted_iota(jnp.int32, (B, N), 0)
    col_ids = jax.lax.broadcasted_iota(jnp.int32, (B, N), 1)
    a = a + dvw_ref[0, 0] * (row_ids == col_ids).astype(jnp.float32)
    a = a * rmask_ref[...]
    a = a * cmask_t_ref[...]
    a = a / (jnp.sum(a, axis=1, keepdims=True) + 1e-8)

    msg = jnp.dot(a, h_full)
    h_new = jnp.tanh(jnp.dot(msg, wg_ref[0]) + bg_ref[0, 0])

    @pl.when(i == 0)
    def _():
        h_scr[pl.ds(row0, B), :] = h_new

    sc = jnp.dot(h_new, wo_ref[...]) + bo_ref[0, 0]
    out_ref[0] = omask_ref[...] * sc


def kernel(ent_feature_embed, ent_adj_ganea, ent_adj_w2v, ent_adj_transE,
           cand_mask_pad, mask, sim_M, diag_val_weight, W_gcn, b_gcn,
           W_out, b_out):
    N, D = ent_feature_embed.shape
    iters = W_gcn.shape[0]
    B = 400
    nb = N // B

    def adj_map(i, b):
        return (jnp.where(i == 0, b, nb - 1), 0)

    grid = (iters, nb)
    out = pl.pallas_call(
        functools.partial(_gcn_body, B=B, N=N),
        grid=grid,
        in_specs=[
            pl.BlockSpec((N, D), lambda i, b: (0, 0)),
            pl.BlockSpec((B, N), adj_map),
            pl.BlockSpec((B, N), adj_map),
            pl.BlockSpec((B, N), adj_map),
            pl.BlockSpec((B, 1), lambda i, b: (b, 0)),
            pl.BlockSpec((1, N), lambda i, b: (0, 0)),
            pl.BlockSpec((D, D), lambda i, b: (0, 0)),
            pl.BlockSpec((1, 1), lambda i, b: (0, 0)),
            pl.BlockSpec((1, D, D), lambda i, b: (i, 0, 0)),
            pl.BlockSpec((1, 1, D), lambda i, b: (i, 0, 0)),
            pl.BlockSpec((D, 1), lambda i, b: (0, 0)),
            pl.BlockSpec((1, 1), lambda i, b: (0, 0)),
            pl.BlockSpec((B, 1), lambda i, b: (b, 0)),
        ],
        out_specs=pl.BlockSpec((1, B, 1), lambda i, b: (i, b, 0)),
        out_shape=jax.ShapeDtypeStruct((iters, N, 1), jnp.float32),
        scratch_shapes=[
            pltpu.VMEM((N, N), jnp.float32),
            pltpu.VMEM((N, D), jnp.float32),
        ],
        compiler_params=pltpu.CompilerParams(
            dimension_semantics=("arbitrary", "arbitrary"),
            vmem_limit_bytes=100 * 1024 * 1024,
        ),
    )(
        ent_feature_embed,
        ent_adj_ganea,
        ent_adj_w2v,
        ent_adj_transE,
        cand_mask_pad,
        cand_mask_pad.reshape(1, N),
        sim_M.reshape(D, D),
        diag_val_weight.reshape(1, 1),
        W_gcn,
        b_gcn.reshape(iters, 1, D),
        W_out,
        b_out.reshape(1, 1),
        mask.reshape(N, 1),
    )
    return out[iters - 1].reshape(mask.shape)


# speedup vs baseline: 6.5861x; 1.1667x over previous
import functools

import jax
import jax.numpy as jnp
from jax.experimental import pallas as pl
from jax.experimental.pallas import tpu as pltpu


def _gcn_body(emb_ref, g_ref, w_ref, t_ref, cmask_ref, sim_ref,
              dvw_ref, wg_ref, bg_ref, wo_ref, bo_ref, omask_ref,
              out_ref, s_scr, h_scr, hn_scr, hm_scr, *, B, N):
    i = pl.program_id(0)
    b = pl.program_id(1)
    row0 = b * B

    @pl.when(b == 0)
    def _():
        @pl.when(i == 0)
        def _():
            h_scr[...] = emb_ref[...]

        @pl.when(i > 0)
        def _():
            h_scr[...] = hn_scr[...]

        hm_scr[...] = cmask_ref[...] * h_scr[...]

    h_full = h_scr[...]
    hb = h_scr[pl.ds(row0, B), :]

    l1 = 0.25 * jnp.dot(hb, sim_ref[...])
    l2 = jax.lax.dot_general(l1, h_full,
                             dimension_numbers=(((1,), (1,)), ((), ())))

    @pl.when(i == 0)
    def _():
        s_scr[pl.ds(row0, B), :] = 0.25 * (g_ref[...] + w_ref[...] + t_ref[...])

    sig = jax.nn.sigmoid(l2 + s_scr[pl.ds(row0, B), :])

    cmask = cmask_ref[...]
    rmask = cmask_ref[pl.ds(row0, B), :]
    dvw_diag = dvw_ref[0, 0] * rmask

    rowsum = jnp.dot(sig, cmask)
    denom = rmask * (rowsum + dvw_diag) + 1e-8
    msg0 = jnp.dot(sig, hm_scr[...])
    msg = rmask * (msg0 + dvw_diag * hb) / denom

    h_new = jnp.tanh(jnp.dot(msg, wg_ref[0]) + bg_ref[0, 0])
    hn_scr[pl.ds(row0, B), :] = h_new

    sc = jnp.dot(h_new, wo_ref[...]) + bo_ref[0, 0]
    out_ref[0] = omask_ref[...] * sc


def kernel(ent_feature_embed, ent_adj_ganea, ent_adj_w2v, ent_adj_transE,
           cand_mask_pad, mask, sim_M, diag_val_weight, W_gcn, b_gcn,
           W_out, b_out):
    N, D = ent_feature_embed.shape
    iters = W_gcn.shape[0]
    B = 400
    nb = N // B

    def adj_map(i, b):
        return (jnp.where(i == 0, b, nb - 1), 0)

    grid = (iters, nb)
    out = pl.pallas_call(
        functools.partial(_gcn_body, B=B, N=N),
        grid=grid,
        in_specs=[
            pl.BlockSpec((N, D), lambda i, b: (0, 0)),
            pl.BlockSpec((B, N), adj_map),
            pl.BlockSpec((B, N), adj_map),
            pl.BlockSpec((B, N), adj_map),
            pl.BlockSpec((N, 1), lambda i, b: (0, 0)),
            pl.BlockSpec((D, D), lambda i, b: (0, 0)),
            pl.BlockSpec((1, 1), lambda i, b: (0, 0)),
            pl.BlockSpec((1, D, D), lambda i, b: (i, 0, 0)),
            pl.BlockSpec((1, 1, D), lambda i, b: (i, 0, 0)),
            pl.BlockSpec((D, 1), lambda i, b: (0, 0)),
            pl.BlockSpec((1, 1), lambda i, b: (0, 0)),
            pl.BlockSpec((B, 1), lambda i, b: (b, 0)),
        ],
        out_specs=pl.BlockSpec((1, B, 1), lambda i, b: (i, b, 0)),
        out_shape=jax.ShapeDtypeStruct((iters, N, 1), jnp.float32),
        scratch_shapes=[
            pltpu.VMEM((N, N), jnp.float32),
            pltpu.VMEM((N, D), jnp.float32),
            pltpu.VMEM((N, D), jnp.float32),
            pltpu.VMEM((N, D), jnp.float32),
        ],
        compiler_params=pltpu.CompilerParams(
            dimension_semantics=("arbitrary", "arbitrary"),
            vmem_limit_bytes=100 * 1024 * 1024,
        ),
    )(
        ent_feature_embed,
        ent_adj_ganea,
        ent_adj_w2v,
        ent_adj_transE,
        cand_mask_pad,
        sim_M.reshape(D, D),
        diag_val_weight.reshape(1, 1),
        W_gcn,
        b_gcn.reshape(iters, 1, D),
        W_out,
        b_out.reshape(1, 1),
        mask.reshape(N, 1),
    )
    return out[iters - 1].reshape(mask.shape)


# speedup vs baseline: 6.7378x; 1.0230x over previous
import functools

import jax
import jax.numpy as jnp
from jax.experimental import pallas as pl
from jax.experimental.pallas import tpu as pltpu


def _gcn_body(emb_ref, g_ref, w_ref, t_ref, cmask_ref, sim_ref,
              dvw_ref, wg_ref, bg_ref, wo_ref, bo_ref, omask_ref,
              out_ref, s_scr, h_scr, hn_scr, hm_scr, ht_scr, hmsum_scr,
              *, B, N):
    i = pl.program_id(0)
    b = pl.program_id(1)
    row0 = b * B

    @pl.when(b == 0)
    def _():
        @pl.when(i == 0)
        def _():
            h_scr[...] = emb_ref[...]

        @pl.when(i > 0)
        def _():
            h_scr[...] = hn_scr[...]

        hm_scr[...] = cmask_ref[...] * h_scr[...]
        ht_scr[...] = h_scr[...].T
        hmsum_scr[...] = jnp.sum(hm_scr[...], axis=0, keepdims=True)

    hb = h_scr[pl.ds(row0, B), :]

    l1 = 0.125 * jnp.dot(hb, sim_ref[...])
    l2 = jnp.dot(l1, ht_scr[...])

    @pl.when(i == 0)
    def _():
        s_scr[pl.ds(row0, B), :] = 0.125 * (g_ref[...] + w_ref[...] + t_ref[...])

    th = jnp.tanh(l2 + s_scr[pl.ds(row0, B), :])

    cmask = cmask_ref[...]
    csum = jnp.sum(cmask)
    rmask = cmask_ref[pl.ds(row0, B), :]
    dvw_diag = dvw_ref[0, 0] * rmask

    rowsum = 0.5 * (jnp.dot(th, cmask) + csum)
    denom = rmask * (rowsum + dvw_diag) + 1e-8
    msg0 = 0.5 * (jnp.dot(th, hm_scr[...]) + hmsum_scr[...])
    msg = rmask * (msg0 + dvw_diag * hb) / denom

    h_new = jnp.tanh(jnp.dot(msg, wg_ref[0]) + bg_ref[0, 0])
    hn_scr[pl.ds(row0, B), :] = h_new

    sc = jnp.dot(h_new, wo_ref[...]) + bo_ref[0, 0]
    out_ref[0] = omask_ref[...] * sc


def kernel(ent_feature_embed, ent_adj_ganea, ent_adj_w2v, ent_adj_transE,
           cand_mask_pad, mask, sim_M, diag_val_weight, W_gcn, b_gcn,
           W_out, b_out):
    N, D = ent_feature_embed.shape
    iters = W_gcn.shape[0]
    B = 400
    nb = N // B

    def adj_map(i, b):
        return (jnp.where(i == 0, b, nb - 1), 0)

    grid = (iters, nb)
    out = pl.pallas_call(
        functools.partial(_gcn_body, B=B, N=N),
        grid=grid,
        in_specs=[
            pl.BlockSpec((N, D), lambda i, b: (0, 0)),
            pl.BlockSpec((B, N), adj_map),
            pl.BlockSpec((B, N), adj_map),
            pl.BlockSpec((B, N), adj_map),
            pl.BlockSpec((N, 1), lambda i, b: (0, 0)),
            pl.BlockSpec((D, D), lambda i, b: (0, 0)),
            pl.BlockSpec((1, 1), lambda i, b: (0, 0)),
            pl.BlockSpec((1, D, D), lambda i, b: (i, 0, 0)),
            pl.BlockSpec((1, 1, D), lambda i, b: (i, 0, 0)),
            pl.BlockSpec((D, 1), lambda i, b: (0, 0)),
            pl.BlockSpec((1, 1), lambda i, b: (0, 0)),
            pl.BlockSpec((B, 1), lambda i, b: (b, 0)),
        ],
        out_specs=pl.BlockSpec((1, B, 1), lambda i, b: (i, b, 0)),
        out_shape=jax.ShapeDtypeStruct((iters, N, 1), jnp.float32),
        scratch_shapes=[
            pltpu.VMEM((N, N), jnp.float32),
            pltpu.VMEM((N, D), jnp.float32),
            pltpu.VMEM((N, D), jnp.float32),
            pltpu.VMEM((N, D), jnp.float32),
            pltpu.VMEM((D, N), jnp.float32),
            pltpu.VMEM((1, D), jnp.float32),
        ],
        compiler_params=pltpu.CompilerParams(
            dimension_semantics=("arbitrary", "arbitrary"),
            vmem_limit_bytes=100 * 1024 * 1024,
        ),
    )(
        ent_feature_embed,
        ent_adj_ganea,
        ent_adj_w2v,
        ent_adj_transE,
        cand_mask_pad,
        sim_M.reshape(D, D),
        diag_val_weight.reshape(1, 1),
        W_gcn,
        b_gcn.reshape(iters, 1, D),
        W_out,
        b_out.reshape(1, 1),
        mask.reshape(N, 1),
    )
    return out[iters - 1].reshape(mask.shape)
